# baseline (speedup 1.0000x reference)
import functools

import jax
import jax.numpy as jnp
from jax import lax
from jax.experimental import pallas as pl
from jax.experimental.pallas import tpu as pltpu
from jax.experimental.pallas import tpu_sc as plsc

_OUT_DIM = 2048
_DPC = 16
_BATCH = 128
_N = _OUT_DIM * _DPC

_NC = 2
_NS = 16
_NW = _NC * _NS
_ROWS_PER_W = _BATCH // _NW

_QPR = 4
_GCHUNK = _OUT_DIM // _QPR
_OUT_LEN = _GCHUNK * _DPC
_IN_LEN = _GCHUNK * (_QPR - 1) + _OUT_LEN
_GBLK = 16
_NBLK = _GCHUNK // _GBLK
_NCHUNK = _ROWS_PER_W * _QPR


def _dk_body(x_hbm, out_hbm, in_v, out_v, sem_in, sem_out):
    wid = lax.axis_index("s") * _NC + lax.axis_index("c")
    row0 = wid * _ROWS_PER_W
    lane = lax.iota(jnp.int32, 16)
    lane15 = lane * 15
    lane16 = lane * 16
    zeros16 = jnp.zeros((16,), jnp.float32)

    def chunk_coords(c):
        row = row0 + c // _QPR
        g0 = (c % _QPR) * _GCHUNK
        return row, g0

    def issue_in(c):
        row, g0 = chunk_coords(c)
        b = c % 2
        pltpu.async_copy(
            x_hbm.at[row, pl.ds(g0 * 15, _IN_LEN)],
            in_v.at[pl.ds(b * _IN_LEN, _IN_LEN)], sem_in.at[b])

    def wait_in(b):
        pltpu.make_async_copy(
            x_hbm.at[0, pl.ds(0, _IN_LEN)],
            in_v.at[pl.ds(0, _IN_LEN)], sem_in.at[b]).wait()

    def issue_out(c):
        row, g0 = chunk_coords(c)
        b = c % 2
        pltpu.async_copy(
            out_v.at[pl.ds(b * _OUT_LEN, _OUT_LEN)],
            out_hbm.at[row, pl.ds(g0 * _DPC, _OUT_LEN)], sem_out.at[b])

    def wait_out(b):
        pltpu.make_async_copy(
            out_v.at[pl.ds(0, _OUT_LEN)],
            out_hbm.at[0, pl.ds(0, _OUT_LEN)], sem_out.at[b]).wait()

    issue_in(0)
    issue_in(1)

    def step(c, _):
        b = c % 2
        ibase = b * _IN_LEN
        obase = b * _OUT_LEN
        _, g0 = chunk_coords(c)
        wait_in(b)
        @pl.when(c >= 2)
        def _():
            wait_out(b)

        @plsc.parallel_loop(0, _OUT_LEN // 16)
        def zero_body(i):
            out_v[pl.ds(obase + i * 16, 16)] = zeros16

        @plsc.parallel_loop(0, _NBLK)
        def blk_body(blk):
            k0 = blk * _GBLK
            wbase = ibase + k0 * 15 + lane15
            vals = [plsc.load_gather(in_v, [wbase + j]) for j in range(_DPC)]
            idxs = [jnp.full((16,), j, jnp.int32) for j in range(_DPC)]
            n = _DPC
            while n > 1:
                half = n // 2
                for i in range(half):
                    v1, v2 = vals[2 * i], vals[2 * i + 1]
                    cond = v2 > v1
                    vals[i] = jnp.where(cond, v2, v1)
                    idxs[i] = jnp.where(cond, idxs[2 * i + 1], idxs[2 * i])
                n = half
            argj = idxs[0]
            out_loc = k0 * 16 + lane16 + argj
            vout = plsc.load_gather(in_v, [out_loc + g0 + ibase])
            plsc.store_scatter(out_v, [out_loc + obase], vout)

        issue_out(c)
        @pl.when(c + 2 < _NCHUNK)
        def _():
            issue_in(c + 2)
        return ()

    lax.fori_loop(0, _NCHUNK, step, ())
    wait_out(0)
    wait_out(1)


@jax.jit
def kernel(x):
    mesh = plsc.VectorSubcoreMesh(core_axis_name="c", subcore_axis_name="s")
    f = functools.partial(
        pl.kernel,
        out_type=jax.ShapeDtypeStruct((_BATCH, _N), jnp.float32),
        mesh=mesh,
        scratch_types=[
            pltpu.VMEM((2 * _IN_LEN,), jnp.float32),
            pltpu.VMEM((2 * _OUT_LEN,), jnp.float32),
            pltpu.SemaphoreType.DMA((2,)),
            pltpu.SemaphoreType.DMA((2,)),
        ],
        compiler_params=pltpu.CompilerParams(needs_layout_passes=False),
    )(_dk_body)
    return f(x)


# speedup vs baseline: 1.6609x; 1.6609x over previous
import functools

import jax
import jax.numpy as jnp
from jax import lax
from jax.experimental import pallas as pl
from jax.experimental.pallas import tpu as pltpu
from jax.experimental.pallas import tpu_sc as plsc

_OUT_DIM = 2048
_DPC = 16
_BATCH = 128
_N = _OUT_DIM * _DPC

_NC = 2
_NS = 16
_NW = _NC * _NS
_ROWS_PER_W = _BATCH // _NW

_QPR = 2
_GCHUNK = _OUT_DIM // _QPR
_OUT_LEN = _GCHUNK * _DPC
_IN_LEN = _GCHUNK * (_QPR - 1) + _OUT_LEN
_GBLK = 16
_NBLK = _GCHUNK // _GBLK
_NCHUNK = _ROWS_PER_W * _QPR


def _dk_body(x_hbm, out_hbm, in0, in1, o0, o1, si0, si1, so0, so1):
    wid = lax.axis_index("s") * _NC + lax.axis_index("c")
    row0 = wid * _ROWS_PER_W
    lane = lax.iota(jnp.int32, 16)
    lane15 = lane * 15
    lane16 = lane * 16
    zeros16 = jnp.zeros((16,), jnp.float32)
    in_bufs = (in0, in1)
    out_bufs = (o0, o1)
    in_sems = (si0, si1)
    out_sems = (so0, so1)

    def chunk_coords(c):
        row = row0 + c // _QPR
        g0 = (c % _QPR) * _GCHUNK
        return row, g0

    def issue_in(c, b):
        row, g0 = chunk_coords(c)
        pltpu.async_copy(
            x_hbm.at[row, pl.ds(g0 * 15, _IN_LEN)], in_bufs[b], in_sems[b])

    def wait_in(b):
        pltpu.make_async_copy(
            x_hbm.at[0, pl.ds(0, _IN_LEN)], in_bufs[b], in_sems[b]).wait()

    def issue_out(c, b):
        row, g0 = chunk_coords(c)
        pltpu.async_copy(
            out_bufs[b], out_hbm.at[row, pl.ds(g0 * _DPC, _OUT_LEN)],
            out_sems[b])

    def wait_out(b):
        pltpu.make_async_copy(
            out_bufs[b], out_hbm.at[0, pl.ds(0, _OUT_LEN)], out_sems[b]).wait()

    def compute(c, b):
        in_v = in_bufs[b]
        out_v = out_bufs[b]
        _, g0 = chunk_coords(c)

        @plsc.parallel_loop(0, _NBLK, unroll=1)
        def blk_body(blk):
            k0 = blk * _GBLK
            wbase = k0 * 15 + lane15
            vals = [plsc.load_gather(in_v, [wbase + j]) for j in range(_DPC)]
            idxs = [jnp.full((16,), j, jnp.int32) for j in range(_DPC)]
            n = _DPC
            while n > 1:
                half = n // 2
                for i in range(half):
                    v1, v2 = vals[2 * i], vals[2 * i + 1]
                    cond = v2 > v1
                    vals[i] = jnp.where(cond, v2, v1)
                    idxs[i] = jnp.where(cond, idxs[2 * i + 1], idxs[2 * i])
                n = half
            argj = idxs[0]
            for i in range(_GBLK):
                out_v[pl.ds((k0 + i) * _DPC, 16)] = zeros16
            out_loc = k0 * 16 + lane16 + argj
            vout = plsc.load_gather(in_v, [out_loc + g0])
            plsc.store_scatter(out_v, [out_loc], vout)

    issue_in(0, 0)
    issue_in(1, 1)

    def step(cc, _):
        for bb in range(2):
            c = cc + bb
            b = bb
            wait_in(b)
            @pl.when(c >= 2)
            def _():
                wait_out(b)
            compute(c, b)
            issue_out(c, b)
            @pl.when(c + 2 < _NCHUNK)
            def _():
                issue_in(c + 2, b)
        return ()

    lax.fori_loop(0, _NCHUNK // 2, lambda i, _: step(i * 2, ()), ())
    wait_out(0)
    wait_out(1)


@jax.jit
def kernel(x):
    mesh = plsc.VectorSubcoreMesh(core_axis_name="c", subcore_axis_name="s")
    f = functools.partial(
        pl.kernel,
        out_type=jax.ShapeDtypeStruct((_BATCH, _N), jnp.float32),
        mesh=mesh,
        scratch_types=[
            pltpu.VMEM((_IN_LEN,), jnp.float32),
            pltpu.VMEM((_IN_LEN,), jnp.float32),
            pltpu.VMEM((_OUT_LEN,), jnp.float32),
            pltpu.VMEM((_OUT_LEN,), jnp.float32),
            pltpu.SemaphoreType.DMA,
            pltpu.SemaphoreType.DMA,
            pltpu.SemaphoreType.DMA,
            pltpu.SemaphoreType.DMA,
        ],
        compiler_params=pltpu.CompilerParams(needs_layout_passes=False),
    )(_dk_body)
    return f(x)
